# baseline (speedup 1.0000x reference)
import functools

import jax
import jax.numpy as jnp
from jax import lax
from jax.experimental import pallas as pl
from jax.experimental.pallas import tpu as pltpu
from jax.experimental.pallas import tpu_sc as plsc

SEQ = 200
BATCH = 4096
DIM = 64

_info = plsc.get_sparse_core_info()
NC, NS = _info.num_cores, _info.num_subcores
NW = NC * NS

N = SEQ * BATCH
B_PER_W = N // NW
CHUNK = 1024
N_CHUNKS = B_PER_W // CHUNK


def _gather_body(table_hbm, idx_hbm, out_hbm, idx_v, rows_v, sem):
    wid = lax.axis_index("s") * NC + lax.axis_index("c")
    base = wid * B_PER_W

    def step(i, carry):
        off = base + i * CHUNK
        pltpu.sync_copy(idx_hbm.at[pl.ds(off, CHUNK)], idx_v)
        pltpu.async_copy(table_hbm.at[idx_v], rows_v, sem).wait()
        pltpu.sync_copy(rows_v, out_hbm.at[pl.ds(off, CHUNK)])
        return carry

    lax.fori_loop(0, N_CHUNKS, step, 0)


_sc_gather = functools.partial(
    pl.kernel,
    out_type=jax.ShapeDtypeStruct((N, DIM), jnp.float32),
    mesh=plsc.VectorSubcoreMesh(core_axis_name="c", subcore_axis_name="s"),
    compiler_params=pltpu.CompilerParams(use_tc_tiling_on_sc=False),
    scratch_types=[
        pltpu.VMEM((CHUNK,), jnp.int32),
        pltpu.VMEM((CHUNK, DIM), jnp.float32),
        pltpu.SemaphoreType.DMA,
    ],
)(_gather_body)


def _mask_body(tok_ref, out_ref):
    out_ref[...] = (tok_ref[...] == 0).astype(jnp.float32).T


_mask_call = pl.pallas_call(
    _mask_body,
    out_shape=jax.ShapeDtypeStruct((BATCH, SEQ), jnp.float32),
)


def kernel(tokens, table):
    tokens = tokens.astype(jnp.int32)
    idx_flat = tokens.reshape(N)
    emb_flat = _sc_gather(table, idx_flat)
    mask = _mask_call(tokens)
    word_emb = emb_flat.reshape(SEQ, BATCH, DIM)
    return (word_emb, mask)


# speedup vs baseline: 1.0092x; 1.0092x over previous
import functools

import jax
import jax.numpy as jnp
from jax import lax
from jax.experimental import pallas as pl
from jax.experimental.pallas import tpu as pltpu
from jax.experimental.pallas import tpu_sc as plsc

SEQ = 200
BATCH = 4096
DIM = 64

_info = plsc.get_sparse_core_info()
NC, NS = _info.num_cores, _info.num_subcores
NW = NC * NS

N = SEQ * BATCH
B_PER_W = N // NW
CHUNK = 400
NBUF = 4
N_CHUNKS = B_PER_W // CHUNK


def _gather_body(table_hbm, idx_hbm, out_hbm, *scratch):
    idx_vs = scratch[0:NBUF]
    rows_vs = scratch[NBUF:2 * NBUF]
    gsems = scratch[2 * NBUF:3 * NBUF]
    osems = scratch[3 * NBUF:4 * NBUF]

    wid = lax.axis_index("s") * NC + lax.axis_index("c")
    base = wid * B_PER_W

    def issue_gather(c, b):
        pltpu.sync_copy(idx_hbm.at[pl.ds(base + c * CHUNK, CHUNK)], idx_vs[b])
        pltpu.async_copy(table_hbm.at[idx_vs[b]], rows_vs[b], gsems[b])

    for c in range(NBUF - 1):
        issue_gather(c, c)

    @pl.loop(0, N_CHUNKS, step=NBUF)
    def _(i):
        for b in range(NBUF):
            j = i + b
            jg = j + NBUF - 1
            bg = (b + NBUF - 1) % NBUF

            @pl.when(jg < N_CHUNKS)
            def _issue():
                if b == 0:
                    @pl.when(i > 0)
                    def _w():
                        pltpu.make_async_copy(
                            rows_vs[bg],
                            out_hbm.at[pl.ds(base, CHUNK)], osems[bg]).wait()
                else:
                    pltpu.make_async_copy(
                        rows_vs[bg],
                        out_hbm.at[pl.ds(base, CHUNK)], osems[bg]).wait()
                issue_gather(jg, bg)

            pltpu.make_async_copy(
                table_hbm.at[idx_vs[b]], rows_vs[b], gsems[b]).wait()
            pltpu.async_copy(
                rows_vs[b], out_hbm.at[pl.ds(base + j * CHUNK, CHUNK)],
                osems[b])

    for b in range(NBUF):
        pltpu.make_async_copy(
            rows_vs[b], out_hbm.at[pl.ds(base, CHUNK)], osems[b]).wait()


_sc_gather = functools.partial(
    pl.kernel,
    out_type=jax.ShapeDtypeStruct((N, DIM), jnp.float32),
    mesh=plsc.VectorSubcoreMesh(core_axis_name="c", subcore_axis_name="s"),
    compiler_params=pltpu.CompilerParams(use_tc_tiling_on_sc=False),
    scratch_types=(
        [pltpu.VMEM((CHUNK,), jnp.int32) for _ in range(NBUF)]
        + [pltpu.VMEM((CHUNK, DIM), jnp.float32) for _ in range(NBUF)]
        + [pltpu.SemaphoreType.DMA for _ in range(2 * NBUF)]
    ),
)(_gather_body)


def _mask_body(tok_ref, out_ref):
    out_ref[...] = (tok_ref[...] == 0).astype(jnp.float32).T


_mask_call = pl.pallas_call(
    _mask_body,
    out_shape=jax.ShapeDtypeStruct((BATCH, SEQ), jnp.float32),
)


def kernel(tokens, table):
    tokens = tokens.astype(jnp.int32)
    idx_flat = tokens.reshape(N)
    emb_flat = _sc_gather(table, idx_flat)
    mask = _mask_call(tokens)
    word_emb = emb_flat.reshape(SEQ, BATCH, DIM)
    return (word_emb, mask)


# speedup vs baseline: 1.3389x; 1.3267x over previous
import functools

import jax
import jax.numpy as jnp
from jax import lax
from jax.experimental import pallas as pl
from jax.experimental.pallas import tpu as pltpu
from jax.experimental.pallas import tpu_sc as plsc

SEQ = 200
BATCH = 4096
DIM = 64
PDIM = 128

_info = plsc.get_sparse_core_info()
NC, NS = _info.num_cores, _info.num_subcores
NW = NC * NS

N = SEQ * BATCH
B_PER_W = N // NW
CHUNK = 256
NBUF = 4
N_CHUNKS = B_PER_W // CHUNK


def _gather_body(table_hbm, idx_hbm, out_hbm, *scratch):
    idx_vs = scratch[0:NBUF]
    rows_vs = scratch[NBUF:2 * NBUF]
    gsems = scratch[2 * NBUF:3 * NBUF]
    osems = scratch[3 * NBUF:4 * NBUF]

    wid = lax.axis_index("s") * NC + lax.axis_index("c")
    base = wid * B_PER_W

    def issue_gather(c, b):
        pltpu.sync_copy(idx_hbm.at[pl.ds(base + c * CHUNK, CHUNK)], idx_vs[b])
        pltpu.async_copy(table_hbm.at[idx_vs[b]], rows_vs[b], gsems[b])

    def out_slice(c):
        off = base + c * CHUNK
        s = off // BATCH
        b0 = off % BATCH
        return out_hbm.at[s, pl.ds(b0, CHUNK), pl.ds(0, DIM)]

    for c in range(NBUF - 1):
        issue_gather(c, c)

    @pl.loop(0, N_CHUNKS, step=NBUF)
    def _(i):
        for b in range(NBUF):
            j = i + b
            jg = j + NBUF - 1
            bg = (b + NBUF - 1) % NBUF

            @pl.when(jg < N_CHUNKS)
            def _issue():
                if b == 0:
                    @pl.when(i > 0)
                    def _w():
                        pltpu.make_async_copy(
                            rows_vs[bg], out_slice(0), osems[bg]).wait()
                else:
                    pltpu.make_async_copy(
                        rows_vs[bg], out_slice(0), osems[bg]).wait()
                issue_gather(jg, bg)

            pltpu.make_async_copy(
                table_hbm.at[idx_vs[b]], rows_vs[b], gsems[b]).wait()
            pltpu.async_copy(rows_vs[b], out_slice(j), osems[b])

    for b in range(NBUF):
        pltpu.make_async_copy(rows_vs[b], out_slice(0), osems[b]).wait()


_sc_gather = functools.partial(
    pl.kernel,
    out_type=jax.ShapeDtypeStruct((SEQ, BATCH, PDIM), jnp.float32),
    mesh=plsc.VectorSubcoreMesh(core_axis_name="c", subcore_axis_name="s"),
    compiler_params=pltpu.CompilerParams(
        use_tc_tiling_on_sc=False, needs_layout_passes=False),
    scratch_types=(
        [pltpu.VMEM((CHUNK,), jnp.int32) for _ in range(NBUF)]
        + [pltpu.VMEM((CHUNK, DIM), jnp.float32) for _ in range(NBUF)]
        + [pltpu.SemaphoreType.DMA for _ in range(2 * NBUF)]
    ),
)(_gather_body)


def _mask_body(tok_ref, out_ref):
    out_ref[...] = (tok_ref[...] == 0).astype(jnp.float32)


_mask_call = pl.pallas_call(
    _mask_body,
    out_shape=jax.ShapeDtypeStruct((SEQ, BATCH), jnp.float32),
)


def kernel(tokens, table):
    tokens = tokens.astype(jnp.int32)
    idx_flat = tokens.reshape(N)
    emb = _sc_gather(table, idx_flat)
    word_emb = emb[..., :DIM].reshape(SEQ, BATCH, DIM)
    mask = _mask_call(tokens).T
    return (word_emb, mask)


# speedup vs baseline: 1.3623x; 1.0175x over previous
import functools

import jax
import jax.numpy as jnp
from jax import lax
from jax.experimental import pallas as pl
from jax.experimental.pallas import tpu as pltpu
from jax.experimental.pallas import tpu_sc as plsc

SEQ = 200
BATCH = 4096
DIM = 64
PDIM = 128

_info = plsc.get_sparse_core_info()
NC, NS = _info.num_cores, _info.num_subcores
NW = NC * NS

N = SEQ * BATCH
B_PER_W = N // NW
CHUNK = 256
NBUF = 4
N_CHUNKS = B_PER_W // CHUNK


def _gather_body(table_hbm, idx_hbm, out_hbm, idx_all, *scratch):
    rows_vs = scratch[0:NBUF]
    gsems = scratch[NBUF:2 * NBUF]
    osems = scratch[2 * NBUF:3 * NBUF]

    wid = lax.axis_index("s") * NC + lax.axis_index("c")
    base = wid * B_PER_W

    pltpu.sync_copy(idx_hbm.at[pl.ds(base, B_PER_W)], idx_all)

    def issue_gather(c, b):
        pltpu.async_copy(
            table_hbm.at[idx_all.at[pl.ds(c * CHUNK, CHUNK)]],
            rows_vs[b], gsems[b])

    def out_slice(c):
        off = base + c * CHUNK
        s = off // BATCH
        b0 = off % BATCH
        return out_hbm.at[s, pl.ds(b0, CHUNK), pl.ds(0, DIM)]

    for c in range(NBUF - 1):
        issue_gather(c, c)

    @pl.loop(0, N_CHUNKS, step=NBUF)
    def _(i):
        for b in range(NBUF):
            j = i + b
            jg = j + NBUF - 1
            bg = (b + NBUF - 1) % NBUF

            @pl.when(jg < N_CHUNKS)
            def _issue():
                if b == 0:
                    @pl.when(i > 0)
                    def _w():
                        pltpu.make_async_copy(
                            rows_vs[bg], out_slice(0), osems[bg]).wait()
                else:
                    pltpu.make_async_copy(
                        rows_vs[bg], out_slice(0), osems[bg]).wait()
                issue_gather(jg, bg)

            pltpu.make_async_copy(
                table_hbm.at[idx_all.at[pl.ds(0, CHUNK)]],
                rows_vs[b], gsems[b]).wait()
            pltpu.async_copy(rows_vs[b], out_slice(j), osems[b])

    for b in range(NBUF):
        pltpu.make_async_copy(rows_vs[b], out_slice(0), osems[b]).wait()


_sc_gather = functools.partial(
    pl.kernel,
    out_type=jax.ShapeDtypeStruct((SEQ, BATCH, PDIM), jnp.float32),
    mesh=plsc.VectorSubcoreMesh(core_axis_name="c", subcore_axis_name="s"),
    compiler_params=pltpu.CompilerParams(
        use_tc_tiling_on_sc=False, needs_layout_passes=False),
    scratch_types=(
        [pltpu.VMEM((B_PER_W,), jnp.int32)]
        + [pltpu.VMEM((CHUNK, DIM), jnp.float32) for _ in range(NBUF)]
        + [pltpu.SemaphoreType.DMA for _ in range(2 * NBUF)]
    ),
)(_gather_body)


def _mask_body(tok_ref, out_ref):
    out_ref[...] = (tok_ref[...] == 0).astype(jnp.float32)


_mask_call = pl.pallas_call(
    _mask_body,
    out_shape=jax.ShapeDtypeStruct((SEQ, BATCH), jnp.float32),
)


def kernel(tokens, table):
    tokens = tokens.astype(jnp.int32)
    idx_flat = tokens.reshape(N)
    emb = _sc_gather(table, idx_flat)
    word_emb = emb[..., :DIM].reshape(SEQ, BATCH, DIM)
    mask = _mask_call(tokens).T
    return (word_emb, mask)


# speedup vs baseline: 1.3659x; 1.0026x over previous
import functools

import jax
import jax.numpy as jnp
from jax import lax
from jax.experimental import pallas as pl
from jax.experimental.pallas import tpu as pltpu
from jax.experimental.pallas import tpu_sc as plsc

SEQ = 200
BATCH = 4096
DIM = 64
PDIM = 128

_info = plsc.get_sparse_core_info()
NC, NS = _info.num_cores, _info.num_subcores
NW = NC * NS

N = SEQ * BATCH
B_PER_W = N // NW
CHUNK = 256
NBUF = 5
N_CHUNKS = B_PER_W // CHUNK


def _gather_body(table_hbm, idx_hbm, out_hbm, idx_all, *scratch):
    rows_vs = scratch[0:NBUF]
    gsems = scratch[NBUF:2 * NBUF]
    osems = scratch[2 * NBUF:3 * NBUF]

    wid = lax.axis_index("s") * NC + lax.axis_index("c")
    base = wid * B_PER_W

    pltpu.sync_copy(idx_hbm.at[pl.ds(base, B_PER_W)], idx_all)

    def issue_gather(c, b):
        pltpu.async_copy(
            table_hbm.at[idx_all.at[pl.ds(c * CHUNK, CHUNK)]],
            rows_vs[b], gsems[b])

    def out_slice(c):
        off = base + c * CHUNK
        s = off // BATCH
        b0 = off % BATCH
        return out_hbm.at[s, pl.ds(b0, CHUNK), pl.ds(0, DIM)]

    for c in range(NBUF - 1):
        issue_gather(c, c)

    @pl.loop(0, N_CHUNKS, step=NBUF)
    def _(i):
        for b in range(NBUF):
            j = i + b
            jg = j + NBUF - 1
            bg = (b + NBUF - 1) % NBUF

            @pl.when(jg < N_CHUNKS)
            def _issue():
                if b == 0:
                    @pl.when(i > 0)
                    def _w():
                        pltpu.make_async_copy(
                            rows_vs[bg], out_slice(0), osems[bg]).wait()
                else:
                    pltpu.make_async_copy(
                        rows_vs[bg], out_slice(0), osems[bg]).wait()
                issue_gather(jg, bg)

            pltpu.make_async_copy(
                table_hbm.at[idx_all.at[pl.ds(0, CHUNK)]],
                rows_vs[b], gsems[b]).wait()
            pltpu.async_copy(rows_vs[b], out_slice(j), osems[b])

    for b in range(NBUF):
        pltpu.make_async_copy(rows_vs[b], out_slice(0), osems[b]).wait()


_sc_gather = functools.partial(
    pl.kernel,
    out_type=jax.ShapeDtypeStruct((SEQ, BATCH, PDIM), jnp.float32),
    mesh=plsc.VectorSubcoreMesh(core_axis_name="c", subcore_axis_name="s"),
    compiler_params=pltpu.CompilerParams(
        use_tc_tiling_on_sc=False, needs_layout_passes=False),
    scratch_types=(
        [pltpu.VMEM((B_PER_W,), jnp.int32)]
        + [pltpu.VMEM((CHUNK, DIM), jnp.float32) for _ in range(NBUF)]
        + [pltpu.SemaphoreType.DMA for _ in range(2 * NBUF)]
    ),
)(_gather_body)


def _mask_body(tok_ref, out_ref):
    out_ref[...] = (tok_ref[...] == 0).astype(jnp.float32)


_mask_call = pl.pallas_call(
    _mask_body,
    out_shape=jax.ShapeDtypeStruct((SEQ, BATCH), jnp.float32),
)


def kernel(tokens, table):
    tokens = tokens.astype(jnp.int32)
    idx_flat = tokens.reshape(N)
    emb = _sc_gather(table, idx_flat)
    word_emb = emb[..., :DIM].reshape(SEQ, BATCH, DIM)
    mask = _mask_call(tokens).T
    return (word_emb, mask)
